# baseline (speedup 1.0000x reference)
import dataclasses
import functools

import jax
import jax.numpy as jnp
from jax import lax
from jax.experimental import pallas as pl
from jax.experimental.pallas import tpu as pltpu
from jax.experimental.pallas import tpu_sc as plsc

HIDDEN = 128
N_TOTAL = 262144
BS = 131072
N_PULL = N_TOTAL - BS

NC = 2
NS = 16
NW = NC * NS
L = 16

NODES_PAD = 1048576
PER_TILE_NODES = NODES_PAD // NW

PUSH_ROWS = BS // HIDDEN
PULL_ROWS = N_PULL // HIDDEN
CHUNK_ROWS = 32

G = 128
GROUPS = N_PULL // NW // G

_mesh = plsc.VectorSubcoreMesh(core_axis_name="c", subcore_axis_name="s")

_cp = pltpu.CompilerParams()
if "needs_layout_passes" in pltpu.CompilerParams.__dataclass_fields__:
    _cp = dataclasses.replace(_cp, needs_layout_passes=False)


def _wid():
    return lax.axis_index("s") * NC + lax.axis_index("c")


@functools.partial(
    pl.kernel,
    out_type=jax.ShapeDtypeStruct((NODES_PAD,), jnp.int32),
    mesh=_mesh,
    scratch_types=[
        pltpu.VMEM((CHUNK_ROWS, HIDDEN), jnp.int32),
        pltpu.VMEM((PER_TILE_NODES,), jnp.int32),
    ],
    compiler_params=_cp,
)
def _build_pos(push_hbm, pos_hbm, idx_v, pos_v):
    wid = _wid()
    lo = wid * PER_TILE_NODES
    hi = lo + PER_TILE_NODES
    neg1 = jnp.full((L,), -1, jnp.int32)
    iota = lax.iota(jnp.int32, L)

    @pl.loop(0, PER_TILE_NODES // L)
    def _(i):
        pos_v[pl.ds(i * L, L)] = neg1

    @pl.loop(0, PUSH_ROWS // CHUNK_ROWS)
    def _(c):
        pltpu.sync_copy(push_hbm.at[pl.ds(c * CHUNK_ROWS, CHUNK_ROWS)], idx_v)

        @pl.loop(0, CHUNK_ROWS)
        def _(r):
            base_j = (c * CHUNK_ROWS + r) * HIDDEN
            for v in range(HIDDEN // L):
                k = idx_v[r, pl.ds(v * L, L)]
                m = (k >= lo) & (k < hi)
                local = jnp.where(m, k - lo, 0)
                jvec = base_j + v * L + iota
                plsc.store_scatter(pos_v, [local], jvec, mask=m)
                cur = plsc.load_gather(pos_v, [local], mask=m)
                m2 = m & (cur < jvec)
                plsc.store_scatter(pos_v, [local], jvec, mask=m2)

    pltpu.sync_copy(pos_v, pos_hbm.at[pl.ds(lo, PER_TILE_NODES)])


@functools.partial(
    pl.kernel,
    out_type=jax.ShapeDtypeStruct((N_TOTAL, HIDDEN), jnp.float32),
    mesh=_mesh,
    scratch_types=[
        pltpu.VMEM((GROUPS, G), jnp.int32),
        pltpu.VMEM((GROUPS, G), jnp.int32),
        pltpu.VMEM((GROUPS, G), jnp.int32),
        pltpu.VMEM((G, HIDDEN), jnp.float32),
        pltpu.VMEM((G, HIDDEN), jnp.float32),
        pltpu.SemaphoreType.DMA,
        pltpu.SemaphoreType.DMA,
    ],
    compiler_params=_cp,
)
def _pull(pos_hbm, pull_hbm, emb_hbm, x_hbm, out_hbm,
          pidx_v, j_v, jsafe_v, erow_v, xrow_v, sem0, sem1):
    wid = _wid()
    pull_base = wid * (N_PULL // NW)
    row_base = wid * (GROUPS * G // G) * G

    xrows_per_tile = BS // NW

    pltpu.sync_copy(pull_hbm.at[pl.ds(wid * GROUPS, GROUPS)], pidx_v)

    @pl.loop(0, GROUPS)
    def _(g):
        pltpu.async_copy(pos_hbm.at[pidx_v.at[g]], j_v.at[g], sem0).wait()
        for v in range(G // L):
            jj = j_v[g, pl.ds(v * L, L)]
            jsafe_v[g, pl.ds(v * L, L)] = jnp.maximum(jj, 0)

    @pl.loop(0, GROUPS)
    def _(g):
        pltpu.async_copy(emb_hbm.at[pidx_v.at[g]], erow_v, sem0).wait()
        pltpu.async_copy(x_hbm.at[jsafe_v.at[g]], xrow_v, sem1).wait()

        @pl.loop(0, G)
        def _(r):
            jb = plsc.load_gather(
                j_v, [jnp.full((L,), g, jnp.int32),
                      jnp.full((L,), r, jnp.int32)])
            take_x = jb >= 0
            for ccol in range(HIDDEN // L):
                sl = pl.ds(ccol * L, L)
                erow_v[r, sl] = jnp.where(take_x, xrow_v[r, sl],
                                          erow_v[r, sl])

        pltpu.sync_copy(
            erow_v, out_hbm.at[pl.ds(BS + pull_base + g * G, G)])

    @pl.loop(0, xrows_per_tile // G)
    def _(g):
        src = pl.ds(wid * xrows_per_tile + g * G, G)
        pltpu.sync_copy(x_hbm.at[src], erow_v)
        pltpu.sync_copy(erow_v, out_hbm.at[src])


def kernel(emb, x, n_id, batch_size):
    bs = BS
    offset = (jnp.asarray(batch_size, dtype=n_id.dtype) - bs)
    push_idx = (n_id[:bs] + offset).reshape(PUSH_ROWS, HIDDEN)
    pull_idx = n_id[bs:].reshape(PULL_ROWS, HIDDEN)
    pos = _build_pos(push_idx)
    out = _pull(pos, pull_idx, emb, x)
    return out
